# baseline (speedup 1.0000x reference)
import jax
import jax.numpy as jnp
from jax import lax
from jax.experimental import pallas as pl
from jax.experimental.pallas import tpu as pltpu
from jax.experimental.pallas import tpu_sc as plsc

N_NODES = 10000
N_FEATS = 9
N_GRAPHS = 512
EMB = 128
ROWS = N_FEATS * EMB

NT = 16
NPT = 640
NPAD = NT * NPT
NW = NPT // 16
ACC = 544
GPT = N_GRAPHS // NT


NV_LAST = N_NODES - (NT - 1) * NPT


def _tw_matvec(tab_ref, w_ref, b_ref, out_ref):
    out_ref[...] = jnp.dot(tab_ref[...], w_ref[...],
                           preferred_element_type=jnp.float32) + b_ref[0, 0] / N_FEATS


def _sc_kernel(x_hbm, bi_hbm, tw_hbm, out_hbm,
               x_v, bi_v, tw_v, sums_v, cnts_v, st_f,
               mg_s, mg_c, out_v, part_sh):
    sid = lax.axis_index("s")
    lane = lax.iota(jnp.int32, 16)

    pltpu.sync_copy(tw_hbm, tw_v)
    n0 = sid * NPT

    @pl.when(sid < NT - 1)
    def _full():
        pltpu.sync_copy(x_hbm.at[pl.ds(n0 * N_FEATS, NPT * N_FEATS)], x_v)
        pltpu.sync_copy(bi_hbm.at[pl.ds(n0, NPT)], bi_v.at[pl.ds(16, NPT)])

    @pl.when(sid == NT - 1)
    def _tail():
        pltpu.sync_copy(x_hbm.at[pl.ds(n0 * N_FEATS, NV_LAST * N_FEATS)],
                        x_v.at[pl.ds(0, NV_LAST * N_FEATS)])
        pltpu.sync_copy(bi_hbm.at[pl.ds(n0, NV_LAST)], bi_v.at[pl.ds(16, NV_LAST)])
        sent = jnp.full((16,), N_GRAPHS, jnp.int32)
        for z in range(NV_LAST, NPT, 16):
            bi_v[pl.ds(16 + z, 16)] = sent

    zeros16 = jnp.zeros((16,), jnp.float32)
    for z in range(ACC // 16):
        sums_v[pl.ds(z * 16, 16)] = zeros16
        cnts_v[pl.ds(z * 16, 16)] = zeros16

    lane9 = lane * N_FEATS

    def win_body(w, _):
        s = jnp.zeros((16,), jnp.float32)
        for f in range(N_FEATS):
            xi = plsc.load_gather(x_v, [lane9 + (w * 16 * N_FEATS + f)])
            s = s + plsc.load_gather(tw_v, [(xi & (EMB - 1)) + f * EMB])
        bidx = bi_v[pl.ds(16 + w * 16, 16)]
        bprev = bi_v[pl.ds(15 + w * 16, 16)]
        bnext = bi_v[pl.ds(17 + w * 16, 16)]

        is_first = (lane == 0) | (bidx != bprev)
        is_last = (lane == 15) | (bidx != bnext)

        csum = plsc.cumsum(s)
        first = plsc.cummax(jnp.where(is_first, lane, 0))
        pb = first - 1

        st_f[...] = csum
        pcs = plsc.load_gather(st_f, [jnp.maximum(pb, 0)])
        pcs = jnp.where(pb < 0, 0.0, pcs)

        seg_sum = csum - pcs
        seg_cnt = (lane - pb).astype(jnp.float32)
        plsc.addupdate_scatter(sums_v, [bidx], seg_sum, mask=is_last)
        plsc.addupdate_scatter(cnts_v, [bidx], seg_cnt, mask=is_last)
        return 0

    lax.fori_loop(0, NW, win_body, 0)

    pltpu.sync_copy(sums_v, part_sh.at[pl.ds(sid * (2 * ACC), ACC)])
    pltpu.sync_copy(cnts_v, part_sh.at[pl.ds(sid * (2 * ACC) + ACC, ACC)])

    plsc.subcore_barrier()

    g0 = sid * GPT
    for t in range(NT):
        base = t * (2 * ACC)
        pltpu.sync_copy(part_sh.at[pl.ds(base + g0, GPT)],
                        mg_s.at[pl.ds(t * GPT, GPT)])
        pltpu.sync_copy(part_sh.at[pl.ds(base + ACC + g0, GPT)],
                        mg_c.at[pl.ds(t * GPT, GPT)])

    for half in range(GPT // 16):
        tot = jnp.zeros((16,), jnp.float32)
        cnt = jnp.zeros((16,), jnp.float32)
        for t in range(NT):
            tot = tot + mg_s[pl.ds(t * GPT + half * 16, 16)]
            cnt = cnt + mg_c[pl.ds(t * GPT + half * 16, 16)]
        z = tot / jnp.maximum(cnt, 1.0)
        out_v[pl.ds(half * 16, 16)] = 1.0 / (1.0 + jnp.exp(-z))

    pltpu.sync_copy(out_v, out_hbm.at[pl.ds(g0, GPT)])


def kernel(x, edge_index, batch_idx, atom_tables, W, b):
    tab = atom_tables.reshape(ROWS, EMB)

    tw = pl.pallas_call(
        _tw_matvec,
        out_shape=jax.ShapeDtypeStruct((ROWS, 1), jnp.float32),
    )(tab, W.astype(jnp.float32),
      b.reshape(1, 1).astype(jnp.float32)).reshape(ROWS)

    mesh = plsc.VectorSubcoreMesh(core_axis_name="c", subcore_axis_name="s",
                                  num_cores=1, num_subcores=NT)
    out = pl.kernel(
        _sc_kernel,
        out_type=jax.ShapeDtypeStruct((N_GRAPHS,), jnp.float32),
        mesh=mesh,
        compiler_params=pltpu.CompilerParams(needs_layout_passes=False),
        scratch_types=[
            pltpu.VMEM((NPT * N_FEATS,), jnp.int32),
            pltpu.VMEM((NPT + 32,), jnp.int32),
            pltpu.VMEM((ROWS,), jnp.float32),
            pltpu.VMEM((ACC,), jnp.float32),
            pltpu.VMEM((ACC,), jnp.float32),
            pltpu.VMEM((16,), jnp.float32),
            pltpu.VMEM((NT * GPT,), jnp.float32),
            pltpu.VMEM((NT * GPT,), jnp.float32),
            pltpu.VMEM((GPT,), jnp.float32),
            pltpu.VMEM_SHARED((NT * 2 * ACC,), jnp.float32),
        ],
    )(x.astype(jnp.int32).reshape(-1), batch_idx.astype(jnp.int32), tw)
    return out.reshape(N_GRAPHS, 1)


# speedup vs baseline: 1.0282x; 1.0282x over previous
import jax
import jax.numpy as jnp
from jax import lax
from jax.experimental import pallas as pl
from jax.experimental.pallas import tpu as pltpu
from jax.experimental.pallas import tpu_sc as plsc

N_NODES = 10000
N_FEATS = 9
N_GRAPHS = 512
EMB = 128
ROWS = N_FEATS * EMB

NT = 16
NPT = 640
NPAD = NT * NPT
NW = NPT // 16
ACC = 544
GPT = N_GRAPHS // NT


NV_LAST = N_NODES - (NT - 1) * NPT


def _tw_matvec(tab_ref, w_ref, b_ref, out_ref):
    out_ref[...] = jnp.dot(tab_ref[...], w_ref[...],
                           preferred_element_type=jnp.float32) + b_ref[0, 0] / N_FEATS


def _sc_kernel(x_hbm, bi_hbm, tw_hbm, out_hbm,
               x_v, bi_v, tw_v, sums_v, cnts_v, st_f,
               mg_s, mg_c, out_v, part_sh):
    sid = lax.axis_index("s")
    lane = lax.iota(jnp.int32, 16)

    pltpu.sync_copy(tw_hbm, tw_v)
    n0 = sid * NPT

    @pl.when(sid < NT - 1)
    def _full():
        for f in range(N_FEATS):
            pltpu.sync_copy(x_hbm.at[pl.ds(f * N_NODES + n0, NPT)],
                            x_v.at[pl.ds(f * NPT, NPT)])
        pltpu.sync_copy(bi_hbm.at[pl.ds(n0, NPT)], bi_v.at[pl.ds(16, NPT)])

    @pl.when(sid == NT - 1)
    def _tail():
        for f in range(N_FEATS):
            pltpu.sync_copy(x_hbm.at[pl.ds(f * N_NODES + n0, NV_LAST)],
                            x_v.at[pl.ds(f * NPT, NV_LAST)])
        pltpu.sync_copy(bi_hbm.at[pl.ds(n0, NV_LAST)], bi_v.at[pl.ds(16, NV_LAST)])
        sent = jnp.full((16,), N_GRAPHS, jnp.int32)
        for z in range(NV_LAST, NPT, 16):
            bi_v[pl.ds(16 + z, 16)] = sent

    zeros16 = jnp.zeros((16,), jnp.float32)
    for z in range(ACC // 16):
        sums_v[pl.ds(z * 16, 16)] = zeros16
        cnts_v[pl.ds(z * 16, 16)] = zeros16

    def win_body(w, _):
        s = jnp.zeros((16,), jnp.float32)
        for f in range(N_FEATS):
            xi = x_v[pl.ds(f * NPT + w * 16, 16)]
            s = s + plsc.load_gather(tw_v, [(xi & (EMB - 1)) + f * EMB])
        bidx = bi_v[pl.ds(16 + w * 16, 16)]
        bprev = bi_v[pl.ds(15 + w * 16, 16)]
        bnext = bi_v[pl.ds(17 + w * 16, 16)]

        is_first = (lane == 0) | (bidx != bprev)
        is_last = (lane == 15) | (bidx != bnext)

        csum = plsc.cumsum(s)
        first = plsc.cummax(jnp.where(is_first, lane, 0))
        pb = first - 1

        st_f[...] = csum
        pcs = plsc.load_gather(st_f, [jnp.maximum(pb, 0)])
        pcs = jnp.where(pb < 0, 0.0, pcs)

        seg_sum = csum - pcs
        seg_cnt = (lane - pb).astype(jnp.float32)
        plsc.addupdate_scatter(sums_v, [bidx], seg_sum, mask=is_last)
        plsc.addupdate_scatter(cnts_v, [bidx], seg_cnt, mask=is_last)
        return 0

    lax.fori_loop(0, NW, win_body, 0)

    pltpu.sync_copy(sums_v, part_sh.at[pl.ds(sid * (2 * ACC), ACC)])
    pltpu.sync_copy(cnts_v, part_sh.at[pl.ds(sid * (2 * ACC) + ACC, ACC)])

    plsc.subcore_barrier()

    g0 = sid * GPT
    for t in range(NT):
        base = t * (2 * ACC)
        pltpu.sync_copy(part_sh.at[pl.ds(base + g0, GPT)],
                        mg_s.at[pl.ds(t * GPT, GPT)])
        pltpu.sync_copy(part_sh.at[pl.ds(base + ACC + g0, GPT)],
                        mg_c.at[pl.ds(t * GPT, GPT)])

    for half in range(GPT // 16):
        tot = jnp.zeros((16,), jnp.float32)
        cnt = jnp.zeros((16,), jnp.float32)
        for t in range(NT):
            tot = tot + mg_s[pl.ds(t * GPT + half * 16, 16)]
            cnt = cnt + mg_c[pl.ds(t * GPT + half * 16, 16)]
        z = tot / jnp.maximum(cnt, 1.0)
        out_v[pl.ds(half * 16, 16)] = 1.0 / (1.0 + jnp.exp(-z))

    pltpu.sync_copy(out_v, out_hbm.at[pl.ds(g0, GPT)])


def kernel(x, edge_index, batch_idx, atom_tables, W, b):
    tab = atom_tables.reshape(ROWS, EMB)

    tw = pl.pallas_call(
        _tw_matvec,
        out_shape=jax.ShapeDtypeStruct((ROWS, 1), jnp.float32),
    )(tab, W.astype(jnp.float32),
      b.reshape(1, 1).astype(jnp.float32)).reshape(ROWS)

    mesh = plsc.VectorSubcoreMesh(core_axis_name="c", subcore_axis_name="s",
                                  num_cores=1, num_subcores=NT)
    out = pl.kernel(
        _sc_kernel,
        out_type=jax.ShapeDtypeStruct((N_GRAPHS,), jnp.float32),
        mesh=mesh,
        compiler_params=pltpu.CompilerParams(needs_layout_passes=False),
        scratch_types=[
            pltpu.VMEM((NPT * N_FEATS,), jnp.int32),
            pltpu.VMEM((NPT + 32,), jnp.int32),
            pltpu.VMEM((ROWS,), jnp.float32),
            pltpu.VMEM((ACC,), jnp.float32),
            pltpu.VMEM((ACC,), jnp.float32),
            pltpu.VMEM((16,), jnp.float32),
            pltpu.VMEM((NT * GPT,), jnp.float32),
            pltpu.VMEM((NT * GPT,), jnp.float32),
            pltpu.VMEM((GPT,), jnp.float32),
            pltpu.VMEM_SHARED((NT * 2 * ACC,), jnp.float32),
        ],
    )(x.astype(jnp.int32).T.reshape(-1), batch_idx.astype(jnp.int32), tw)
    return out.reshape(N_GRAPHS, 1)


# speedup vs baseline: 1.1573x; 1.1256x over previous
import jax
import jax.numpy as jnp
from jax import lax
from jax.experimental import pallas as pl
from jax.experimental.pallas import tpu as pltpu
from jax.experimental.pallas import tpu_sc as plsc

N_NODES = 10000
N_FEATS = 9
N_GRAPHS = 512
EMB = 128
ROWS = N_FEATS * EMB

NT = 16
NPT = 640
NPAD = NT * NPT
NW = NPT // 16
ACC = 544
GPT = N_GRAPHS // NT


def _tw_matvec(tab_ref, w_ref, b_ref, out_ref):
    out_ref[...] = jnp.dot(tab_ref[...], w_ref[...],
                           preferred_element_type=jnp.float32) + b_ref[0, 0] / N_FEATS


def _sc_kernel(x_hbm, bi_hbm, tw_hbm, out_hbm,
               x_v, bi_v, tw_v, sums_v, cnts_v, st_f,
               mg_s, mg_c, out_v, part_sh):
    sid = lax.axis_index("s")
    lane = lax.iota(jnp.int32, 16)

    pltpu.sync_copy(tw_hbm, tw_v)
    n0 = sid * NPT

    pltpu.sync_copy(x_hbm.at[pl.ds(sid * (NPT * N_FEATS), NPT * N_FEATS)], x_v)
    pltpu.sync_copy(bi_hbm.at[pl.ds(n0, NPT)], bi_v.at[pl.ds(16, NPT)])

    zeros16 = jnp.zeros((16,), jnp.float32)
    for z in range(ACC // 16):
        sums_v[pl.ds(z * 16, 16)] = zeros16
        cnts_v[pl.ds(z * 16, 16)] = zeros16

    def win_body(w, _):
        s = jnp.zeros((16,), jnp.float32)
        for f in range(N_FEATS):
            xi = x_v[pl.ds(f * NPT + w * 16, 16)]
            s = s + plsc.load_gather(tw_v, [(xi & (EMB - 1)) + f * EMB])
        bidx = bi_v[pl.ds(16 + w * 16, 16)]
        bprev = bi_v[pl.ds(15 + w * 16, 16)]
        bnext = bi_v[pl.ds(17 + w * 16, 16)]

        is_first = (lane == 0) | (bidx != bprev)
        is_last = (lane == 15) | (bidx != bnext)

        csum = plsc.cumsum(s)
        first = plsc.cummax(jnp.where(is_first, lane, 0))
        pb = first - 1

        st_f[...] = csum
        pcs = plsc.load_gather(st_f, [jnp.maximum(pb, 0)])
        pcs = jnp.where(pb < 0, 0.0, pcs)

        seg_sum = csum - pcs
        seg_cnt = (lane - pb).astype(jnp.float32)
        plsc.addupdate_scatter(sums_v, [bidx], seg_sum, mask=is_last)
        plsc.addupdate_scatter(cnts_v, [bidx], seg_cnt, mask=is_last)
        return 0

    lax.fori_loop(0, NW, win_body, 0)

    pltpu.sync_copy(sums_v, part_sh.at[pl.ds(sid * (2 * ACC), ACC)])
    pltpu.sync_copy(cnts_v, part_sh.at[pl.ds(sid * (2 * ACC) + ACC, ACC)])

    plsc.subcore_barrier()

    g0 = sid * GPT
    for t in range(NT):
        base = t * (2 * ACC)
        pltpu.sync_copy(part_sh.at[pl.ds(base + g0, GPT)],
                        mg_s.at[pl.ds(t * GPT, GPT)])
        pltpu.sync_copy(part_sh.at[pl.ds(base + ACC + g0, GPT)],
                        mg_c.at[pl.ds(t * GPT, GPT)])

    for half in range(GPT // 16):
        tot = jnp.zeros((16,), jnp.float32)
        cnt = jnp.zeros((16,), jnp.float32)
        for t in range(NT):
            tot = tot + mg_s[pl.ds(t * GPT + half * 16, 16)]
            cnt = cnt + mg_c[pl.ds(t * GPT + half * 16, 16)]
        z = tot / jnp.maximum(cnt, 1.0)
        out_v[pl.ds(half * 16, 16)] = 1.0 / (1.0 + jnp.exp(-z))

    pltpu.sync_copy(out_v, out_hbm.at[pl.ds(g0, GPT)])


def kernel(x, edge_index, batch_idx, atom_tables, W, b):
    tab = atom_tables.reshape(ROWS, EMB)

    xp = jnp.pad(x.astype(jnp.int32), ((0, NPAD - N_NODES), (0, 0)))
    xt = xp.T.reshape(N_FEATS, NT, NPT).transpose(1, 0, 2).reshape(-1)
    bi = jnp.pad(batch_idx.astype(jnp.int32), (0, NPAD - N_NODES),
                 constant_values=N_GRAPHS)

    tw = pl.pallas_call(
        _tw_matvec,
        out_shape=jax.ShapeDtypeStruct((ROWS, 1), jnp.float32),
    )(tab, W.astype(jnp.float32),
      b.reshape(1, 1).astype(jnp.float32)).reshape(ROWS)

    mesh = plsc.VectorSubcoreMesh(core_axis_name="c", subcore_axis_name="s",
                                  num_cores=1, num_subcores=NT)
    out = pl.kernel(
        _sc_kernel,
        out_type=jax.ShapeDtypeStruct((N_GRAPHS,), jnp.float32),
        mesh=mesh,
        compiler_params=pltpu.CompilerParams(needs_layout_passes=False),
        scratch_types=[
            pltpu.VMEM((NPT * N_FEATS,), jnp.int32),
            pltpu.VMEM((NPT + 32,), jnp.int32),
            pltpu.VMEM((ROWS,), jnp.float32),
            pltpu.VMEM((ACC,), jnp.float32),
            pltpu.VMEM((ACC,), jnp.float32),
            pltpu.VMEM((16,), jnp.float32),
            pltpu.VMEM((NT * GPT,), jnp.float32),
            pltpu.VMEM((NT * GPT,), jnp.float32),
            pltpu.VMEM((GPT,), jnp.float32),
            pltpu.VMEM_SHARED((NT * 2 * ACC,), jnp.float32),
        ],
    )(xt, bi, tw)
    return out.reshape(N_GRAPHS, 1)
